# Initial kernel scaffold; baseline (speedup 1.0000x reference)
import jax
import jax.numpy as jnp
from jax.experimental import pallas as pl


def kernel(x):
    raise NotImplementedError("write your pallas kernel here")


# speedup vs baseline: 1.0038x; 1.0038x over previous
import functools

import jax
import jax.numpy as jnp
import numpy as np
from jax.experimental import pallas as pl
from jax.experimental.pallas import tpu as pltpu


def _perm_matrix(width, f):
    wq = width // f
    k = np.arange(width)
    src = (k % wq) * f + (k // wq)
    m = np.zeros((width, width), np.float32)
    m[src, k] = 1.0
    return m


def _squeeze_body(x_ref, p_ref, o_ref):
    rblk, hblk, fw_total = x_ref.shape
    W = fw_total // 2
    Wo = W // 2
    P = p_ref[...]
    xb = x_ref[...].astype(jnp.bfloat16)
    for fh in range(2):
        rows = xb[:, :, fh * W:(fh + 1) * W].reshape(rblk * hblk, W)
        perm = jnp.dot(rows, P, preferred_element_type=jnp.float32)
        perm = perm.reshape(rblk, hblk, W)
        for fw in range(2):
            o_ref[:, fh * 2 + fw, :, :] = perm[:, :, fw * Wo:(fw + 1) * Wo]


def kernel(x):
    B, C, H, W = x.shape
    f = 2
    Ho, Wo = H // f, W // f
    N = B * C
    xv = x.reshape(N, Ho, f * W)
    P = jnp.asarray(_perm_matrix(W, f), jnp.bfloat16)

    rblk = 8
    grid = (N // rblk,)

    out = pl.pallas_call(
        _squeeze_body,
        out_shape=jax.ShapeDtypeStruct((N, f * f, Ho, Wo), x.dtype),
        grid=grid,
        in_specs=[
            pl.BlockSpec((rblk, Ho, f * W), lambda g: (g, 0, 0)),
            pl.BlockSpec((W, W), lambda g: (0, 0)),
        ],
        out_specs=pl.BlockSpec((rblk, f * f, Ho, Wo), lambda g: (g, 0, 0, 0)),
        compiler_params=pltpu.CompilerParams(
            dimension_semantics=("parallel",),
            vmem_limit_bytes=36 * 2**20),
        cost_estimate=pl.CostEstimate(
            flops=N * H * W * W, transcendentals=0,
            bytes_accessed=2 * x.size * x.dtype.itemsize),
    )(xv, P)
    return out.reshape(B, C * f * f, Ho, Wo)


# speedup vs baseline: 1.0662x; 1.0621x over previous
import functools

import jax
import jax.numpy as jnp
import numpy as np
from jax.experimental import pallas as pl
from jax.experimental.pallas import tpu as pltpu


def _perm_matrix(width, f):
    wq = width // f
    k = np.arange(width)
    src = (k % wq) * f + (k // wq)
    m = np.zeros((width, width), np.float32)
    m[src, k] = 1.0
    return m


def _squeeze_body(x_ref, p_ref, o_ref):
    rblk, hblk, fw_total = x_ref.shape
    W = fw_total // 2
    Wo = W // 2
    P = p_ref[...]
    xb = x_ref[...].astype(jnp.bfloat16)
    for fh in range(2):
        rows = xb[:, :, fh * W:(fh + 1) * W].reshape(rblk * hblk, W)
        perm = jnp.dot(rows, P, preferred_element_type=jnp.float32)
        perm = perm.reshape(rblk, hblk, W)
        for fw in range(2):
            o_ref[:, fh * 2 + fw, :, :] = perm[:, :, fw * Wo:(fw + 1) * Wo]


def kernel(x):
    B, C, H, W = x.shape
    f = 2
    Ho, Wo = H // f, W // f
    N = B * C
    xv = x.reshape(N, Ho, f * W)
    P = jnp.asarray(_perm_matrix(W, f), jnp.bfloat16)

    rblk = 24
    grid = (N // rblk,)

    out = pl.pallas_call(
        _squeeze_body,
        out_shape=jax.ShapeDtypeStruct((N, f * f, Ho, Wo), x.dtype),
        grid=grid,
        in_specs=[
            pl.BlockSpec((rblk, Ho, f * W), lambda g: (g, 0, 0)),
            pl.BlockSpec((W, W), lambda g: (0, 0)),
        ],
        out_specs=pl.BlockSpec((rblk, f * f, Ho, Wo), lambda g: (g, 0, 0, 0)),
        compiler_params=pltpu.CompilerParams(
            dimension_semantics=("parallel",),
            vmem_limit_bytes=58 * 2**20),
        cost_estimate=pl.CostEstimate(
            flops=N * H * W * W, transcendentals=0,
            bytes_accessed=2 * x.size * x.dtype.itemsize),
    )(xv, P)
    return out.reshape(B, C * f * f, Ho, Wo)


# speedup vs baseline: 1.0700x; 1.0035x over previous
import functools

import jax
import jax.numpy as jnp
import numpy as np
from jax.experimental import pallas as pl
from jax.experimental.pallas import tpu as pltpu


def _perm_matrix(width, f):
    wq = width // f
    k = np.arange(width)
    src = (k % wq) * f + (k // wq)
    m = np.zeros((width, width), np.float32)
    m[src, k] = 1.0
    return m


def _squeeze_body(x_ref, p_ref, o_ref):
    rblk, hblk, fw_total = x_ref.shape
    W = fw_total // 2
    Wo = W // 2
    P = p_ref[...]
    xb = x_ref[...].astype(jnp.bfloat16)
    for fh in range(2):
        rows = xb[:, :, fh * W:(fh + 1) * W].reshape(rblk * hblk, W)
        perm = jnp.dot(rows, P, preferred_element_type=jnp.float32)
        perm = perm.reshape(rblk, hblk, W)
        for fw in range(2):
            o_ref[:, fh * 2 + fw, :, :] = perm[:, :, fw * Wo:(fw + 1) * Wo]


def kernel(x):
    B, C, H, W = x.shape
    f = 2
    Ho, Wo = H // f, W // f
    N = B * C
    xv = x.reshape(N, Ho, f * W)
    P = jnp.asarray(_perm_matrix(W, f), jnp.bfloat16)

    rblk = 32
    grid = (N // rblk,)

    out = pl.pallas_call(
        _squeeze_body,
        out_shape=jax.ShapeDtypeStruct((N, f * f, Ho, Wo), x.dtype),
        grid=grid,
        in_specs=[
            pl.BlockSpec((rblk, Ho, f * W), lambda g: (g, 0, 0)),
            pl.BlockSpec((W, W), lambda g: (0, 0)),
        ],
        out_specs=pl.BlockSpec((rblk, f * f, Ho, Wo), lambda g: (g, 0, 0, 0)),
        compiler_params=pltpu.CompilerParams(
            dimension_semantics=("parallel",),
            vmem_limit_bytes=58 * 2**20),
        cost_estimate=pl.CostEstimate(
            flops=N * H * W * W, transcendentals=0,
            bytes_accessed=2 * x.size * x.dtype.itemsize),
    )(xv, P)
    return out.reshape(B, C * f * f, Ho, Wo)
